# baseline (speedup 1.0000x reference)
import functools

import jax
import jax.numpy as jnp
from jax import lax
from jax.experimental import pallas as pl
from jax.experimental.pallas import tpu as pltpu
from jax.experimental.pallas import tpu_sc as plsc

N_NODES = 10000
N_EDGES = 320000
D = 128
EPS = 1e-5

NC = 2
NS = 16
NW = NC * NS
E_PER_W = N_EDGES // NW
CHUNK = 88
NCHUNK = 114
NPAD = 10240
ROWS_PER_TILE = NPAD // NS
ZROWS = CHUNK


def _spmm_body(xw_hbm, src_hbm, dst_hbm, zeros_hbm, out_hbm,
               src_v, dst_v, rows0, acc_sh, sem_g0):
    c = lax.axis_index("c")
    s = lax.axis_index("s")
    wid = c * NS + s

    pltpu.sync_copy(src_hbm.at[wid], src_v)
    pltpu.sync_copy(dst_hbm.at[wid], dst_v)

    pltpu.sync_copy(zeros_hbm, rows0)
    row0 = s * ROWS_PER_TILE
    zoff = 0
    zchunks = [ZROWS] * (ROWS_PER_TILE // ZROWS)
    if ROWS_PER_TILE % ZROWS:
        zchunks.append(ROWS_PER_TILE % ZROWS)
    for zr in zchunks:
        pltpu.sync_copy(rows0.at[pl.ds(0, zr)],
                        acc_sh.at[pl.ds(row0 + zoff, zr)])
        zoff += zr
    plsc.subcore_barrier()

    def body(i, carry):
        pltpu.async_copy(xw_hbm.at[src_v.at[i]], rows0, sem_g0).wait()
        pltpu.sync_copy(rows0, acc_sh.at[dst_v.at[i]], add=True)
        return carry

    lax.fori_loop(0, NCHUNK, body, 0)
    plsc.subcore_barrier()

    pltpu.sync_copy(acc_sh.at[pl.ds(row0, ROWS_PER_TILE)],
                    out_hbm.at[c, pl.ds(row0, ROWS_PER_TILE)])


@functools.cache
def _make_spmm_sc():
    mesh = plsc.VectorSubcoreMesh(core_axis_name="c", subcore_axis_name="s",
                                  num_cores=NC, num_subcores=NS)
    return pl.kernel(
        _spmm_body,
        mesh=mesh,
        out_type=jax.ShapeDtypeStruct((NC, NPAD, D), jnp.float32),
        scratch_types=[
            pltpu.VMEM((NCHUNK, CHUNK), jnp.int32),
            pltpu.VMEM((NCHUNK, CHUNK), jnp.int32),
            pltpu.VMEM((CHUNK, D), jnp.float32),
            pltpu.VMEM_SHARED((NPAD, D), jnp.float32),
            pltpu.SemaphoreType.DMA,
        ],
    )


def _spmm_sc(xw, src, dst, zeros):
    return _make_spmm_sc()(xw, src, dst, zeros)


def _mm2_body(h_ref, ws_ref, wn_ref, s_ref, n_ref):
    h = h_ref[...]
    dn = (((1,), (1,)), ((), ()))
    s_ref[...] = lax.dot_general(h, ws_ref[...], dn,
                                 preferred_element_type=jnp.float32)
    n_ref[...] = lax.dot_general(h, wn_ref[...], dn,
                                 preferred_element_type=jnp.float32)


_mm2 = pl.pallas_call(
    _mm2_body,
    out_shape=(jax.ShapeDtypeStruct((N_NODES, D), jnp.float32),
               jax.ShapeDtypeStruct((N_NODES, D), jnp.float32)),
)


def _mid_body(s_ref, a0_ref, a1_ref, g_ref, b_ref, ws_ref, wn_ref,
              s_out_ref, n_out_ref):
    h = (s_ref[...] + a0_ref[pl.ds(0, N_NODES), :]
         + a1_ref[pl.ds(0, N_NODES), :])
    mu = jnp.mean(h, axis=0, keepdims=True)
    var = jnp.mean((h - mu) ** 2, axis=0, keepdims=True)
    h = (h - mu) * lax.rsqrt(var + EPS) * g_ref[...] + b_ref[...]
    h = jnp.maximum(h, 0.0)
    dn = (((1,), (1,)), ((), ()))
    s_out_ref[...] = lax.dot_general(h, ws_ref[...], dn,
                                     preferred_element_type=jnp.float32)
    n_out_ref[...] = lax.dot_general(h, wn_ref[...], dn,
                                     preferred_element_type=jnp.float32)


_mid = pl.pallas_call(
    _mid_body,
    out_shape=(jax.ShapeDtypeStruct((N_NODES, D), jnp.float32),
               jax.ShapeDtypeStruct((N_NODES, D), jnp.float32)),
)


def _final_body(s_ref, a0_ref, a1_ref, o_ref):
    h = (s_ref[...] + a0_ref[pl.ds(0, N_NODES), :]
         + a1_ref[pl.ds(0, N_NODES), :])
    m = jnp.max(h, axis=1, keepdims=True)
    e = jnp.exp(h - m)
    lse = jnp.log(jnp.sum(e, axis=1, keepdims=True)) + m
    o_ref[...] = h - lse


_final = pl.pallas_call(
    _final_body,
    out_shape=jax.ShapeDtypeStruct((N_NODES, D), jnp.float32),
)


def kernel(x, edge_index, W_self1, W_neigh1, bn1_gamma, bn1_beta,
           W_self2, W_neigh2, bn2_gamma, bn2_beta, W_self3, W_neigh3):
    npad = NCHUNK * CHUNK - E_PER_W
    src = jnp.pad(edge_index[0].astype(jnp.int32).reshape(NW, E_PER_W),
                  ((0, 0), (0, npad))).reshape(NW, NCHUNK, CHUNK)
    pad_dst = jnp.broadcast_to(
        N_NODES + jnp.arange(npad, dtype=jnp.int32), (NW, npad))
    dst = jnp.concatenate(
        [edge_index[1].astype(jnp.int32).reshape(NW, E_PER_W), pad_dst],
        axis=1).reshape(NW, NCHUNK, CHUNK)
    zeros = jnp.zeros((ZROWS, D), jnp.float32)
    g1 = bn1_gamma.reshape(1, D)
    b1 = bn1_beta.reshape(1, D)
    g2 = bn2_gamma.reshape(1, D)
    b2 = bn2_beta.reshape(1, D)

    s1, n1 = _mm2(x, W_self1, W_neigh1)
    agg = _spmm_sc(n1, src, dst, zeros)
    s2, n2 = _mid(s1, agg[0], agg[1], g1, b1, W_self2, W_neigh2)
    agg = _spmm_sc(n2, src, dst, zeros)
    s3, n3 = _mid(s2, agg[0], agg[1], g2, b2, W_self3, W_neigh3)
    agg = _spmm_sc(n3, src, dst, zeros)
    return _final(s3, agg[0], agg[1])


# speedup vs baseline: 1.0921x; 1.0921x over previous
import functools

import jax
import jax.numpy as jnp
from jax import lax
from jax.experimental import pallas as pl
from jax.experimental.pallas import tpu as pltpu
from jax.experimental.pallas import tpu_sc as plsc

N_NODES = 10000
N_EDGES = 320000
D = 128
EPS = 1e-5

NC = 2
NS = 16
NW = NC * NS
E_PER_W = N_EDGES // NW
CHUNK = 80
NCHUNK = 125
NPAD = 10240
ROWS_PER_TILE = NPAD // NS
ZROWS = CHUNK


def _spmm_body(xw_hbm, src_hbm, dst_hbm, zeros_hbm, out_hbm,
               src_v, dst_v, rows0, acc_sh, sem_g0):
    c = lax.axis_index("c")
    s = lax.axis_index("s")
    wid = c * NS + s

    pltpu.sync_copy(src_hbm.at[wid], src_v)
    pltpu.sync_copy(dst_hbm.at[wid], dst_v)

    pltpu.sync_copy(zeros_hbm, rows0)
    row0 = s * ROWS_PER_TILE
    for j in range(ROWS_PER_TILE // ZROWS):
        pltpu.sync_copy(rows0, acc_sh.at[pl.ds(row0 + j * ZROWS, ZROWS)])
    plsc.subcore_barrier()

    def body(i, carry):
        pltpu.async_copy(xw_hbm.at[src_v.at[i]], rows0, sem_g0).wait()
        pltpu.sync_copy(rows0, acc_sh.at[dst_v.at[i]], add=True)
        return carry

    lax.fori_loop(0, NCHUNK, body, 0)
    plsc.subcore_barrier()

    pltpu.sync_copy(acc_sh.at[pl.ds(row0, ROWS_PER_TILE)],
                    out_hbm.at[c, pl.ds(row0, ROWS_PER_TILE)])


@functools.cache
def _make_spmm_sc():
    mesh = plsc.VectorSubcoreMesh(core_axis_name="c", subcore_axis_name="s",
                                  num_cores=NC, num_subcores=NS)
    return pl.kernel(
        _spmm_body,
        mesh=mesh,
        out_type=jax.ShapeDtypeStruct((NC, NPAD, D), jnp.float32),
        scratch_types=[
            pltpu.VMEM((NCHUNK, CHUNK), jnp.int32),
            pltpu.VMEM((NCHUNK, CHUNK), jnp.int32),
            pltpu.VMEM((CHUNK, D), jnp.float32),
            pltpu.VMEM_SHARED((NPAD, D), jnp.float32),
            pltpu.SemaphoreType.DMA,
        ],
    )


def _spmm_sc(xw, src, dst, zeros):
    return _make_spmm_sc()(xw, src, dst, zeros)


def _mm2_body(h_ref, ws_ref, wn_ref, s_ref, n_ref):
    h = h_ref[...]
    dn = (((1,), (1,)), ((), ()))
    s_ref[...] = lax.dot_general(h, ws_ref[...], dn,
                                 preferred_element_type=jnp.float32)
    n_ref[...] = lax.dot_general(h, wn_ref[...], dn,
                                 preferred_element_type=jnp.float32)


_mm2 = pl.pallas_call(
    _mm2_body,
    out_shape=(jax.ShapeDtypeStruct((N_NODES, D), jnp.float32),
               jax.ShapeDtypeStruct((N_NODES, D), jnp.float32)),
)


def _mid_body(s_ref, a0_ref, a1_ref, g_ref, b_ref, ws_ref, wn_ref,
              s_out_ref, n_out_ref):
    h = (s_ref[...] + a0_ref[pl.ds(0, N_NODES), :]
         + a1_ref[pl.ds(0, N_NODES), :])
    mu = jnp.mean(h, axis=0, keepdims=True)
    var = jnp.mean((h - mu) ** 2, axis=0, keepdims=True)
    h = (h - mu) * lax.rsqrt(var + EPS) * g_ref[...] + b_ref[...]
    h = jnp.maximum(h, 0.0)
    dn = (((1,), (1,)), ((), ()))
    s_out_ref[...] = lax.dot_general(h, ws_ref[...], dn,
                                     preferred_element_type=jnp.float32)
    n_out_ref[...] = lax.dot_general(h, wn_ref[...], dn,
                                     preferred_element_type=jnp.float32)


_mid = pl.pallas_call(
    _mid_body,
    out_shape=(jax.ShapeDtypeStruct((N_NODES, D), jnp.float32),
               jax.ShapeDtypeStruct((N_NODES, D), jnp.float32)),
)


def _final_body(s_ref, a0_ref, a1_ref, o_ref):
    h = (s_ref[...] + a0_ref[pl.ds(0, N_NODES), :]
         + a1_ref[pl.ds(0, N_NODES), :])
    m = jnp.max(h, axis=1, keepdims=True)
    e = jnp.exp(h - m)
    lse = jnp.log(jnp.sum(e, axis=1, keepdims=True)) + m
    o_ref[...] = h - lse


_final = pl.pallas_call(
    _final_body,
    out_shape=jax.ShapeDtypeStruct((N_NODES, D), jnp.float32),
)


def kernel(x, edge_index, W_self1, W_neigh1, bn1_gamma, bn1_beta,
           W_self2, W_neigh2, bn2_gamma, bn2_beta, W_self3, W_neigh3):
    src = edge_index[0].astype(jnp.int32).reshape(NW, NCHUNK, CHUNK)
    dst = edge_index[1].astype(jnp.int32).reshape(NW, NCHUNK, CHUNK)
    zeros = jnp.zeros((ZROWS, D), jnp.float32)
    g1 = bn1_gamma.reshape(1, D)
    b1 = bn1_beta.reshape(1, D)
    g2 = bn2_gamma.reshape(1, D)
    b2 = bn2_beta.reshape(1, D)

    s1, n1 = _mm2(x, W_self1, W_neigh1)
    agg = _spmm_sc(n1, src, dst, zeros)
    s2, n2 = _mid(s1, agg[0], agg[1], g1, b1, W_self2, W_neigh2)
    agg = _spmm_sc(n2, src, dst, zeros)
    s3, n3 = _mid(s2, agg[0], agg[1], g2, b2, W_self3, W_neigh3)
    agg = _spmm_sc(n3, src, dst, zeros)
    return _final(s3, agg[0], agg[1])


# speedup vs baseline: 1.1246x; 1.0298x over previous
import functools

import jax
import jax.numpy as jnp
from jax import lax
from jax.experimental import pallas as pl
from jax.experimental.pallas import tpu as pltpu
from jax.experimental.pallas import tpu_sc as plsc

N_NODES = 10000
N_EDGES = 320000
D = 128
EPS = 1e-5

NC = 2
NS = 16
NW = NC * NS
E_PER_W = N_EDGES // NW
CHUNK = 80
NCHUNK = 125
NPAD = 10240
ROWS_PER_TILE = NPAD // NS
ZROWS = CHUNK


def _spmm_body(xw_hbm, src_hbm, dst_hbm, zeros_hbm, out_hbm,
               src_v, dst_v, rows0, acc_sh, sem_g0):
    c = lax.axis_index("c")
    s = lax.axis_index("s")
    wid = c * NS + s

    pltpu.sync_copy(src_hbm.at[wid], src_v)
    pltpu.sync_copy(dst_hbm.at[wid], dst_v)

    pltpu.sync_copy(zeros_hbm, rows0)
    row0 = s * ROWS_PER_TILE
    for j in range(ROWS_PER_TILE // ZROWS):
        pltpu.sync_copy(rows0, acc_sh.at[pl.ds(row0 + j * ZROWS, ZROWS)])
    plsc.subcore_barrier()

    def body(i, carry):
        pltpu.async_copy(xw_hbm.at[src_v.at[i]], rows0, sem_g0).wait()
        pltpu.sync_copy(rows0, acc_sh.at[dst_v.at[i]], add=True)
        return carry

    lax.fori_loop(0, NCHUNK, body, 0)
    plsc.subcore_barrier()

    pltpu.sync_copy(acc_sh.at[pl.ds(row0, ROWS_PER_TILE)],
                    out_hbm.at[c, pl.ds(row0, ROWS_PER_TILE)])


@functools.cache
def _make_spmm_sc():
    mesh = plsc.VectorSubcoreMesh(core_axis_name="c", subcore_axis_name="s",
                                  num_cores=NC, num_subcores=NS)
    return pl.kernel(
        _spmm_body,
        mesh=mesh,
        out_type=jax.ShapeDtypeStruct((NC, NPAD, D), jnp.float32),
        scratch_types=[
            pltpu.VMEM((NCHUNK, CHUNK), jnp.int32),
            pltpu.VMEM((NCHUNK, CHUNK), jnp.int32),
            pltpu.VMEM((CHUNK, D), jnp.float32),
            pltpu.VMEM_SHARED((NPAD, D), jnp.float32),
            pltpu.SemaphoreType.DMA,
        ],
    )


def _spmm_sc(xw, src, dst, zeros):
    return _make_spmm_sc()(xw, src, dst, zeros)


def _mm2_body(h_ref, ws_ref, wn_ref, s_ref, n_ref):
    h = h_ref[...]
    dn = (((1,), (1,)), ((), ()))
    s_ref[...] = lax.dot_general(h, ws_ref[...], dn,
                                 preferred_element_type=jnp.float32)
    n_ref[...] = lax.dot_general(h, wn_ref[...], dn,
                                 preferred_element_type=jnp.float32)


_mm2 = pl.pallas_call(
    _mm2_body,
    out_shape=(jax.ShapeDtypeStruct((N_NODES, D), jnp.float32),
               jax.ShapeDtypeStruct((N_NODES, D), jnp.float32)),
)


def _mid_body(s_ref, agg_ref, g_ref, b_ref, ws_ref, wn_ref,
              s_out_ref, n_out_ref):
    h = (s_ref[...] + agg_ref[0, pl.ds(0, N_NODES), :]
         + agg_ref[1, pl.ds(0, N_NODES), :])
    mu = jnp.mean(h, axis=0, keepdims=True)
    var = jnp.mean((h - mu) ** 2, axis=0, keepdims=True)
    h = (h - mu) * lax.rsqrt(var + EPS) * g_ref[...] + b_ref[...]
    h = jnp.maximum(h, 0.0)
    dn = (((1,), (1,)), ((), ()))
    s_out_ref[...] = lax.dot_general(h, ws_ref[...], dn,
                                     preferred_element_type=jnp.float32)
    n_out_ref[...] = lax.dot_general(h, wn_ref[...], dn,
                                     preferred_element_type=jnp.float32)


_mid = pl.pallas_call(
    _mid_body,
    out_shape=(jax.ShapeDtypeStruct((N_NODES, D), jnp.float32),
               jax.ShapeDtypeStruct((N_NODES, D), jnp.float32)),
)


def _final_body(s_ref, agg_ref, o_ref):
    h = (s_ref[...] + agg_ref[0, pl.ds(0, N_NODES), :]
         + agg_ref[1, pl.ds(0, N_NODES), :])
    m = jnp.max(h, axis=1, keepdims=True)
    e = jnp.exp(h - m)
    lse = jnp.log(jnp.sum(e, axis=1, keepdims=True)) + m
    o_ref[...] = h - lse


_final = pl.pallas_call(
    _final_body,
    out_shape=jax.ShapeDtypeStruct((N_NODES, D), jnp.float32),
)


def kernel(x, edge_index, W_self1, W_neigh1, bn1_gamma, bn1_beta,
           W_self2, W_neigh2, bn2_gamma, bn2_beta, W_self3, W_neigh3):
    src = edge_index[0].astype(jnp.int32).reshape(NW, NCHUNK, CHUNK)
    dst = edge_index[1].astype(jnp.int32).reshape(NW, NCHUNK, CHUNK)
    zeros = jnp.zeros((ZROWS, D), jnp.float32)
    g1 = bn1_gamma.reshape(1, D)
    b1 = bn1_beta.reshape(1, D)
    g2 = bn2_gamma.reshape(1, D)
    b2 = bn2_beta.reshape(1, D)

    s1, n1 = _mm2(x, W_self1, W_neigh1)
    agg = _spmm_sc(n1, src, dst, zeros)
    s2, n2 = _mid(s1, agg, g1, b1, W_self2, W_neigh2)
    agg = _spmm_sc(n2, src, dst, zeros)
    s3, n3 = _mid(s2, agg, g2, b2, W_self3, W_neigh3)
    agg = _spmm_sc(n3, src, dst, zeros)
    return _final(s3, agg)
